# baseline (speedup 1.0000x reference)
import jax
import jax.numpy as jnp
from jax.experimental import pallas as pl
from jax.experimental.pallas import tpu as pltpu

VOCAB = 8192
D = 256
T_TILE = 256
COMMITMENT_COST = 0.25


def _vq_body(x_ref, x2_ref, w_ref, o_ref, loss_ref):
    i = pl.program_id(0)
    n = pl.num_programs(0)
    x = x_ref[...]
    w = w_ref[...]
    cross = jax.lax.dot_general(
        x, w, (((1,), (1,)), ((), ())),
        preferred_element_type=jnp.float32)
    d2 = jnp.sqrt(jnp.maximum(x2_ref[...] - 2.0 * cross, 0.0))
    m = jnp.min(d2, axis=1, keepdims=True)
    iota = jax.lax.broadcasted_iota(jnp.int32, (T_TILE, VOCAB), 1)
    idx = jnp.min(jnp.where(d2 == m, iota, VOCAB), axis=1, keepdims=True)
    onehot = (iota == idx).astype(jnp.float32)
    q = jax.lax.dot_general(
        onehot, w, (((1,), (0,)), ((), ())),
        precision=jax.lax.Precision.HIGHEST,
        preferred_element_type=jnp.float32)
    o_ref[...] = x + (q - x)
    s = jnp.sum((q - x) ** 2)

    @pl.when(i == 0)
    def _init():
        loss_ref[0, 0] = 0.0

    loss_ref[0, 0] += s

    @pl.when(i == n - 1)
    def _finish():
        loss_ref[0, 0] = loss_ref[0, 0] * (
            (1.0 + COMMITMENT_COST) / (n * T_TILE * D))


def kernel(x, W):
    B, T, Dx = x.shape
    x2 = jnp.sum(x * x, axis=-1, keepdims=True)
    xf = x.reshape(B * T, Dx)
    x2f = x2.reshape(B * T, 1)
    grid = (B * T) // T_TILE
    out, loss = pl.pallas_call(
        _vq_body,
        grid=(grid,),
        in_specs=[
            pl.BlockSpec((T_TILE, D), lambda i: (i, 0)),
            pl.BlockSpec((T_TILE, 1), lambda i: (i, 0)),
            pl.BlockSpec((VOCAB, D), lambda i: (0, 0)),
        ],
        out_specs=[
            pl.BlockSpec((T_TILE, D), lambda i: (i, 0)),
            pl.BlockSpec(memory_space=pltpu.SMEM, block_shape=(1, 1),
                         index_map=lambda i: (0, 0)),
        ],
        out_shape=[
            jax.ShapeDtypeStruct((B * T, D), jnp.float32),
            jax.ShapeDtypeStruct((1, 1), jnp.float32),
        ],
    )(xf, x2f, W)
    return out.reshape(B, T, Dx), loss.reshape(())


# speedup vs baseline: 1.9900x; 1.9900x over previous
import functools

import jax
import jax.numpy as jnp
from jax import lax
from jax.experimental import pallas as pl
from jax.experimental.pallas import tpu as pltpu
from jax.experimental.pallas import tpu_sc as plsc

VOCAB = 8192
D = 256
T_TILE = 256
COMMITMENT_COST = 0.25


def _argmin_body(x_ref, x2_ref, w_ref, idx_ref):
    x = x_ref[...]
    w = w_ref[...]
    cross = lax.dot_general(
        x, w, (((1,), (1,)), ((), ())),
        preferred_element_type=jnp.float32)
    dist = jnp.sqrt(jnp.maximum(x2_ref[...] - 2.0 * cross, 0.0))
    m = jnp.min(dist, axis=1, keepdims=True)
    iota = lax.broadcasted_iota(jnp.int32, (T_TILE, VOCAB), 1)
    idx_ref[...] = jnp.min(jnp.where(dist == m, iota, VOCAB), axis=1,
                           keepdims=True)


def _argmin_stage(xf, x2f, W):
    n_tok = xf.shape[0]
    grid = n_tok // T_TILE
    return pl.pallas_call(
        _argmin_body,
        grid=(grid,),
        in_specs=[
            pl.BlockSpec((T_TILE, D), lambda i: (i, 0)),
            pl.BlockSpec((T_TILE, 1), lambda i: (i, 0)),
            pl.BlockSpec((VOCAB, D), lambda i: (0, 0)),
        ],
        out_specs=pl.BlockSpec((T_TILE, 1), lambda i: (i, 0)),
        out_shape=jax.ShapeDtypeStruct((n_tok, 1), jnp.int32),
    )(xf, x2f, W)


def _make_gather(n_tok):
    info = plsc.get_sparse_core_info()
    nc, ns, nl = info.num_cores, info.num_subcores, info.num_lanes
    nw = nc * ns
    b_per_w = n_tok // nw
    n_chunks = b_per_w // 128
    mesh = plsc.VectorSubcoreMesh(core_axis_name="c", subcore_axis_name="s")

    @functools.partial(
        pl.kernel, mesh=mesh,
        out_type=jax.ShapeDtypeStruct((n_tok, D), jnp.float32),
        scratch_types=[
            pltpu.VMEM((n_chunks, 128), jnp.int32),
            pltpu.VMEM((b_per_w, D), jnp.float32),
            pltpu.SemaphoreType.DMA,
        ],
    )
    def gather(idx_hbm, table_hbm, out_hbm, idx_v, rows_v, sem):
        wid = lax.axis_index("s") * nc + lax.axis_index("c")
        pltpu.sync_copy(idx_hbm.at[pl.ds(wid * n_chunks, n_chunks)], idx_v)
        copies = []
        for j in range(n_chunks):
            copies.append(pltpu.async_copy(
                table_hbm.at[idx_v.at[j]],
                rows_v.at[pl.ds(j * 128, 128)], sem))
        for c in copies:
            c.wait()
        pltpu.sync_copy(rows_v, out_hbm.at[pl.ds(wid * b_per_w, b_per_w)])

    return gather


_ST_TILE = 1024


def _st_body(x_ref, q_ref, o_ref, loss_ref):
    i = pl.program_id(0)
    n = pl.num_programs(0)
    x = x_ref[...]
    q = q_ref[...]
    o_ref[...] = x + (q - x)
    s = jnp.sum((q - x) ** 2)

    @pl.when(i == 0)
    def _init():
        loss_ref[0, 0] = 0.0

    loss_ref[0, 0] += s

    @pl.when(i == n - 1)
    def _finish():
        loss_ref[0, 0] = loss_ref[0, 0] * (
            (1.0 + COMMITMENT_COST) / (n * _ST_TILE * D))


def _st_stage(xf, q):
    n_tok = xf.shape[0]
    grid = n_tok // _ST_TILE
    return pl.pallas_call(
        _st_body,
        grid=(grid,),
        in_specs=[
            pl.BlockSpec((_ST_TILE, D), lambda i: (i, 0)),
            pl.BlockSpec((_ST_TILE, D), lambda i: (i, 0)),
        ],
        out_specs=[
            pl.BlockSpec((_ST_TILE, D), lambda i: (i, 0)),
            pl.BlockSpec(memory_space=pltpu.SMEM, block_shape=(1, 1),
                         index_map=lambda i: (0, 0)),
        ],
        out_shape=[
            jax.ShapeDtypeStruct((n_tok, D), jnp.float32),
            jax.ShapeDtypeStruct((1, 1), jnp.float32),
        ],
    )(xf, q)


def kernel(x, W):
    B, T, Dx = x.shape
    n_tok = B * T
    x2 = jnp.sum(x * x, axis=-1, keepdims=True)
    xf = x.reshape(n_tok, Dx)
    x2f = x2.reshape(n_tok, 1)
    idx = _argmin_stage(xf, x2f, W)
    idx128 = idx.reshape(n_tok // 128, 128)
    q = _make_gather(n_tok)(idx128, W)
    out, loss = _st_stage(xf, q)
    return out.reshape(B, T, Dx), loss.reshape(())


# speedup vs baseline: 2.3782x; 1.1951x over previous
import functools

import jax
import jax.numpy as jnp
from jax import lax
from jax.experimental import pallas as pl
from jax.experimental.pallas import tpu as pltpu
from jax.experimental.pallas import tpu_sc as plsc

VOCAB = 8192
D = 256
T_TILE = 256
COMMITMENT_COST = 0.25


def _succ(z):
    return lax.bitcast_convert_type(
        lax.bitcast_convert_type(z, jnp.int32) + 1, jnp.float32)


def _pred(z):
    return lax.bitcast_convert_type(
        lax.bitcast_convert_type(z, jnp.int32) - 1, jnp.float32)


def _argmin_body(x_ref, x2_ref, w_ref, idx_ref, loss_ref):
    i = pl.program_id(0)
    n = pl.num_programs(0)
    x = x_ref[...]
    w = w_ref[...]
    t = lax.dot_general(
        x + x, w, (((1,), (1,)), ((), ())),
        preferred_element_type=jnp.float32)
    d2 = x2_ref[...] - t
    m2 = jnp.maximum(jnp.min(d2, axis=1, keepdims=True), 0.0)
    s = jnp.sqrt(m2)
    ulp = _succ(s) - s
    t_hi = s * s + s * ulp
    for _ in range(3):
        t_hi = jnp.where(jnp.sqrt(t_hi) > s, _pred(t_hi), t_hi)
    for _ in range(3):
        t_nxt = _succ(t_hi)
        t_hi = jnp.where(jnp.sqrt(t_nxt) <= s, t_nxt, t_hi)
    iota = lax.broadcasted_iota(jnp.int32, (T_TILE, VOCAB), 1)
    idx_ref[...] = jnp.min(jnp.where(d2 <= t_hi, iota, VOCAB), axis=1,
                           keepdims=True)
    part = jnp.sum(m2)

    @pl.when(i == 0)
    def _init():
        loss_ref[0, 0] = 0.0

    loss_ref[0, 0] += part

    @pl.when(i == n - 1)
    def _finish():
        loss_ref[0, 0] = loss_ref[0, 0] * (
            (1.0 + COMMITMENT_COST) / (n * T_TILE * D))


def _argmin_stage(xf, x2f, W):
    n_tok = xf.shape[0]
    grid = n_tok // T_TILE
    return pl.pallas_call(
        _argmin_body,
        grid=(grid,),
        in_specs=[
            pl.BlockSpec((T_TILE, D), lambda i: (i, 0)),
            pl.BlockSpec((T_TILE, 1), lambda i: (i, 0)),
            pl.BlockSpec((VOCAB, D), lambda i: (0, 0)),
        ],
        out_specs=[
            pl.BlockSpec((T_TILE, 1), lambda i: (i, 0)),
            pl.BlockSpec(memory_space=pltpu.SMEM, block_shape=(1, 1),
                         index_map=lambda i: (0, 0)),
        ],
        out_shape=[
            jax.ShapeDtypeStruct((n_tok, 1), jnp.int32),
            jax.ShapeDtypeStruct((1, 1), jnp.float32),
        ],
    )(xf, x2f, W)


def _make_gather(n_tok):
    info = plsc.get_sparse_core_info()
    nc, ns, nl = info.num_cores, info.num_subcores, info.num_lanes
    nw = nc * ns
    b_per_w = n_tok // nw
    n_chunks = b_per_w // 128
    mesh = plsc.VectorSubcoreMesh(core_axis_name="c", subcore_axis_name="s")

    @functools.partial(
        pl.kernel, mesh=mesh,
        out_type=jax.ShapeDtypeStruct((n_tok, D), jnp.float32),
        scratch_types=[
            pltpu.VMEM((n_chunks, 128), jnp.int32),
            pltpu.VMEM((b_per_w, D), jnp.float32),
            pltpu.SemaphoreType.DMA,
        ],
    )
    def gather(idx_hbm, table_hbm, out_hbm, idx_v, rows_v, sem):
        wid = lax.axis_index("s") * nc + lax.axis_index("c")
        pltpu.sync_copy(idx_hbm.at[pl.ds(wid * n_chunks, n_chunks)], idx_v)
        copies = []
        for j in range(n_chunks):
            copies.append(pltpu.async_copy(
                table_hbm.at[idx_v.at[j]],
                rows_v.at[pl.ds(j * 128, 128)], sem))
        for c in copies:
            c.wait()
        pltpu.sync_copy(rows_v, out_hbm.at[pl.ds(wid * b_per_w, b_per_w)])

    return gather


_ST_TILE = 1024


def _st_body(x_ref, q_ref, o_ref):
    x = x_ref[...]
    q = q_ref[...]
    o_ref[...] = x + (q - x)


def _st_stage(xf, q):
    n_tok = xf.shape[0]
    grid = n_tok // _ST_TILE
    return pl.pallas_call(
        _st_body,
        grid=(grid,),
        in_specs=[
            pl.BlockSpec((_ST_TILE, D), lambda i: (i, 0)),
            pl.BlockSpec((_ST_TILE, D), lambda i: (i, 0)),
        ],
        out_specs=pl.BlockSpec((_ST_TILE, D), lambda i: (i, 0)),
        out_shape=jax.ShapeDtypeStruct((n_tok, D), jnp.float32),
    )(xf, q)


def kernel(x, W):
    B, T, Dx = x.shape
    n_tok = B * T
    x2 = jnp.sum(x * x, axis=-1, keepdims=True)
    xf = x.reshape(n_tok, Dx)
    x2f = x2.reshape(n_tok, 1)
    idx, loss = _argmin_stage(xf, x2f, W)
    idx128 = idx.reshape(n_tok // 128, 128)
    q = _make_gather(n_tok)(idx128, W)
    out = _st_stage(xf, q)
    return out.reshape(B, T, Dx), loss.reshape(())
